# baseline (speedup 1.0000x reference)
import functools

import numpy as np
import jax
import jax.numpy as jnp
from jax import lax
from jax.experimental import pallas as pl
from jax.experimental.pallas import tpu as pltpu
from jax.experimental.pallas import tpu_sc as plsc

_NC = 80
_IMG = 640
_NB = 32
_NT = 1024
_N = 15360
_ANCH = np.array(
    [[10.0, 13.0], [16.0, 30.0], [33.0, 23.0], [30.0, 61.0], [62.0, 45.0],
     [59.0, 119.0], [116.0, 90.0], [156.0, 198.0], [373.0, 326.0]],
    dtype=np.float32)
_HWS = [(80, 80), (40, 40), (20, 20)]
_CELL_BASE = [0, _NB * 3 * 6400, _NB * 3 * 6400 + _NB * 3 * 1600]
_DTOT = _NB * 3 * (6400 + 1600 + 400)
_SENT = 4.0e6

_NW = 32
_DCH = _DTOT // _NW
_CK = 128
_PERMS = [(0, 2, 3, 1), (0, 2, 3, 1), (2, 3, 0, 1)]


def _repack_body(*refs, bh, W):
    p_ref, tab_ref, obj_ref = refs[0], refs[-2], refs[-1]
    i = pl.program_id(0)

    @pl.when(i == 0)
    def _():
        obj_ref[...] = jnp.zeros_like(obj_ref)

    x = p_ref[0]
    x2 = x.reshape(bh * W, 255)
    xp = jnp.concatenate(
        [x2, jnp.zeros((bh * W, 1), jnp.float32)], axis=1)
    b = lax.bitcast_convert_type(xp, jnp.int32)
    r = (b + 0x7FFF + ((b >> 16) & 1)) >> 16
    tab_ref[...] = (r[:, :128] & 0xFFFF) | (r[:, 128:256] << 16)
    s = (jnp.sum(jnp.logaddexp(0.0, x2[:, 4:5])) +
         jnp.sum(jnp.logaddexp(0.0, x2[:, 89:90])) +
         jnp.sum(jnp.logaddexp(0.0, x2[:, 174:175])))
    r = lax.broadcasted_iota(jnp.int32, (8, 128), 0)
    c = lax.broadcasted_iota(jnp.int32, (8, 128), 1)
    obj_ref[...] += jnp.where((r == 0) & (c == 0), s, 0.0)


def _repack(p_cl, s, chain=None):
    d0, d1, d2 = p_cl.shape[0], p_cl.shape[1], p_cl.shape[2]
    R = d0 * d1 * d2
    extra = [] if chain is None else [chain]
    return pl.pallas_call(
        functools.partial(_repack_body, bh=d1, W=d2),
        grid=(d0,),
        out_shape=[jax.ShapeDtypeStruct((R, 128), jnp.int32),
                   jax.ShapeDtypeStruct((8, 128), jnp.float32)],
        in_specs=([pl.BlockSpec((1, d1, d2, 255), lambda i: (i, 0, 0, 0))] +
                  [pl.BlockSpec((8, 128), lambda i: (0, 0))
                   for _ in extra]),
        out_specs=[pl.BlockSpec((d1 * d2, 128), lambda i: (i, 0)),
                   pl.BlockSpec((8, 128), lambda i: (0, 0))],
    )(p_cl, *extra)


def _prep_body(tt_ref, aux_ref, meta_ref, *, H, W, anchors, cell_base, border):
    col = lax.broadcasted_iota(jnp.int32, (1, _N), 1)
    a = (col // _NT) % 3
    o = col // (3 * _NT)

    def tiled(r):
        row = tt_ref[r:r + 1, :]
        return jnp.concatenate([row] * 15, axis=1)

    bi = tiled(0)
    cls_f = tiled(1)
    gx = tiled(2) * W
    gy = tiled(3) * H
    gw = tiled(4) * W
    gh = tiled(5) * H

    af = a.astype(jnp.float32)
    aw = jnp.where(af == 0.0, anchors[0, 0],
                   jnp.where(af == 1.0, anchors[1, 0], anchors[2, 0]))
    ah = jnp.where(af == 0.0, anchors[0, 1],
                   jnp.where(af == 1.0, anchors[1, 1], anchors[2, 1]))
    rw = gw / aw
    rh = gh / ah
    fitf = jnp.where(
        jnp.maximum(jnp.maximum(rw, 1.0 / rw), jnp.maximum(rh, 1.0 / rh)) < 4.0,
        1.0, 0.0)
    gxi = W - gx
    gyi = H - gy

    def near(u):
        return jnp.where(u % 1.0 < 0.5, 1.0, 0.0) * jnp.where(u > 1.0, 1.0, 0.0)

    jk0, jk1, lm0, lm1 = near(gx), near(gy), near(gxi), near(gyi)
    jmf = jnp.where(o == 0, 1.0,
                    jnp.where(o == 1, jk0,
                              jnp.where(o == 2, jk1,
                                        jnp.where(o == 3, lm0, lm1))))
    validf = jmf * fitf
    valid = validf > 0.5
    ox = jnp.where(o == 1, 1.0, jnp.where(o == 3, -1.0, 0.0))
    oy = jnp.where(o == 2, 1.0, jnp.where(o == 4, -1.0, 0.0))
    gi0 = (gx - ox).astype(jnp.int32)
    gj0 = (gy - oy).astype(jnp.int32)
    gi = jnp.clip(gi0, 0, W - 1)
    gj = jnp.clip(gj0, 0, H - 1)
    bii = bi.astype(jnp.int32)
    clsi = cls_f.astype(jnp.int32)

    if border:
        rowidx = (gj * W + gi) * _NB + bii
    else:
        rowidx = (bii * H + gj) * W + gi
    zero = jnp.zeros((1, _N), jnp.int32)
    aux_ref[...] = jnp.concatenate(
        [rowidx, clsi, zero, zero, zero, zero, zero, zero], axis=0)

    cellf = jnp.where(
        valid,
        (((bii * 3 + a) * H + gj) * W + gi + cell_base).astype(jnp.float32),
        _SENT)
    meta_ref[...] = jnp.concatenate(
        [gx - gi0.astype(jnp.float32), gy - gj0.astype(jnp.float32),
         gw, gh, validf, cellf,
         jnp.broadcast_to(aw, (1, _N)), jnp.broadcast_to(ah, (1, _N))],
        axis=0)


def _prep(tt_pad, s):
    H, W = _HWS[s]
    stride = _IMG // W
    anchors = _ANCH[s * 3:(s + 1) * 3] / stride
    return pl.pallas_call(
        functools.partial(_prep_body, H=H, W=W, anchors=anchors,
                          cell_base=_CELL_BASE[s], border=(s == 2)),
        out_shape=[
            jax.ShapeDtypeStruct((8, _N), jnp.int32),
            jax.ShapeDtypeStruct((8, _N), jnp.float32),
        ],
        in_specs=[pl.BlockSpec((8, _NT), lambda: (0, 0))],
        out_specs=[pl.BlockSpec((8, _N), lambda: (0, 0)),
                   pl.BlockSpec((8, _N), lambda: (0, 0))],
    )(tt_pad)


def _sc_gather_body(tab, aux, clsout, hot,
                    rowbuf, clsbuf, databuf, hotbuf, gsem):
    wid = lax.axis_index("s") * 2 + lax.axis_index("c")
    lane = lax.iota(jnp.int32, 16)
    trips = (120 - wid + 31) // 32

    def chunk(t_, _):
        ci = wid + 32 * t_
        a_ = (ci // 8) % 3
        col0 = pl.multiple_of(ci * _CK, 128)
        pltpu.sync_copy(aux.at[0, pl.ds(col0, _CK)], rowbuf)
        pltpu.sync_copy(aux.at[1, pl.ds(col0, _CK)], clsbuf)
        pltpu.async_copy(tab.at[rowbuf], databuf, gsem).wait()

        def getchan(q, c):
            w = plsc.load_gather(databuf, [q, c % 128])
            bits = jnp.where(c >= 128,
                             w & jnp.int32(-65536), w << 16)
            return plsc.bitcast(bits, jnp.float32)

        def sub(i, _):
            q = i * 16 + lane
            for ch in range(5):
                c = a_ * 85 + jnp.full((16,), ch, jnp.int32)
                hotbuf[ch, pl.ds(i * 16, 16)] = getchan(q, c)
            cv = clsbuf[pl.ds(i * 16, 16)]
            hotbuf[5, pl.ds(i * 16, 16)] = getchan(q, a_ * 85 + 5 + cv)
            return _

        lax.fori_loop(0, _CK // 16, sub, 0)
        pltpu.sync_copy(databuf, clsout.at[pl.ds(col0, _CK), :])
        pltpu.sync_copy(hotbuf, hot.at[:, pl.ds(col0, _CK)])
        return _

    lax.fori_loop(0, trips, chunk, 0)


def _sc_dedup_body(m0, m1, m2, win, cellbuf, dense, winbuf):
    wid = lax.axis_index("s") * 2 + lax.axis_index("c")
    metas = (m0, m1, m2)
    lane = lax.iota(jnp.int32, 16)

    def ms(i, _):
        dense[pl.ds(i * 16, 16)] = jnp.full((16,), -1, jnp.int32)
        return _

    lax.fori_loop(0, _DCH // 16, ms, 0, unroll=4)

    wbase = wid * _DCH
    for s_ in range(3):
        pltpu.sync_copy(metas[s_].at[5, :], cellbuf)

        def p1b(i, _):
            c = cellbuf[pl.ds(i * 16, 16)].astype(jnp.int32) - wbase
            m = (c >= 0) & (c < _DCH)
            cs = jnp.where(m, c, 0)
            plsc.store_scatter(dense, [cs], i * 16 + lane, mask=m)
            return _

        lax.fori_loop(0, _N // 16, p1b, 0, unroll=4)

        def p2b(i, _):
            c = cellbuf[pl.ds(i * 16, 16)].astype(jnp.int32) - wbase
            m = (c >= 0) & (c < _DCH)
            cs = jnp.where(m, c, 0)
            w = plsc.load_gather(dense, [cs], mask=m)
            isw = m & (w == i * 16 + lane)
            winbuf[pl.ds(i * 16, 16)] = jnp.where(isw, 1.0, 0.0)
            return _

        lax.fori_loop(0, _N // 16, p2b, 0, unroll=4)
        pltpu.sync_copy(winbuf, win.at[wid, pl.ds(s_ * _N, _N)])


def _sc_mesh():
    return plsc.VectorSubcoreMesh(core_axis_name="c", subcore_axis_name="s",
                                  num_cores=2, num_subcores=16)


def _sc_g2d_body(tab, aux, m0, m1, m2, clsout, hot, win,
                 rowbuf, clsbuf, databuf, hotbuf, cellbuf, dense, winbuf,
                 gsem):
    _sc_gather_body(tab, aux, clsout, hot, rowbuf, clsbuf, databuf, hotbuf,
                    gsem)
    _sc_dedup_body(m0, m1, m2, win, cellbuf, dense, winbuf)


def _sc_g2d(tab, aux, metas):
    f = pl.kernel(
        _sc_g2d_body,
        out_type=[
            jax.ShapeDtypeStruct((_N, 128), jnp.int32),
            jax.ShapeDtypeStruct((8, _N), jnp.float32),
            jax.ShapeDtypeStruct((_NW, 3 * _N), jnp.float32),
        ],
        mesh=_sc_mesh(),
        scratch_types=[
            pltpu.VMEM((_CK,), jnp.int32),
            pltpu.VMEM((_CK,), jnp.int32),
            pltpu.VMEM((_CK, 128), jnp.int32),
            pltpu.VMEM((8, _CK), jnp.float32),
            pltpu.VMEM((_N,), jnp.float32),
            pltpu.VMEM((_DCH,), jnp.int32),
            pltpu.VMEM((_N,), jnp.float32),
            pltpu.SemaphoreType.DMA,
        ],
        compiler_params=pltpu.CompilerParams(needs_layout_passes=False),
    )
    return f(tab, aux, *metas)


def _sc_gather(tab, aux):
    f = pl.kernel(
        _sc_gather_body,
        out_type=[
            jax.ShapeDtypeStruct((_N, 128), jnp.int32),
            jax.ShapeDtypeStruct((8, _N), jnp.float32),
        ],
        mesh=_sc_mesh(),
        scratch_types=[
            pltpu.VMEM((_CK,), jnp.int32),
            pltpu.VMEM((_CK,), jnp.int32),
            pltpu.VMEM((_CK, 128), jnp.int32),
            pltpu.VMEM((8, _CK), jnp.float32),
            pltpu.SemaphoreType.DMA,
        ],
        compiler_params=pltpu.CompilerParams(needs_layout_passes=False),
    )
    return f(tab, aux)


def _atan_pos(x):
    inv = x > 1.0
    y = jnp.where(inv, 1.0 / x, x)
    z = y * y
    p = y * (0.9998660 + z * (-0.3302995 + z * (0.1801410 + z *
             (-0.0851330 + z * 0.0208351))))
    return jnp.where(inv, (np.pi / 2) - p, p)


def _math_body(c0, c1, c2, h0, h1, h2, m0, m1, m2, w0, w1, w2, out_ref):
    i = pl.program_id(0)

    @pl.when(i == 0)
    def _():
        out_ref[...] = jnp.zeros_like(out_ref)

    a_dyn = i % 3
    acc = jnp.zeros((8, 128), jnp.float32)
    r_i = lax.broadcasted_iota(jnp.int32, (8, 128), 0)
    c_i = lax.broadcasted_iota(jnp.int32, (8, 128), 1)
    for s_, (cb, hot, mt, w) in enumerate(((c0, h0, m0, w0), (c1, h1, m1, w1),
                                           (c2, h2, m2, w2))):
        tbx = mt[0:1, :]
        tby = mt[1:2, :]
        tbw = mt[2:3, :]
        tbh = mt[3:4, :]
        valid = mt[4:5, :]
        aw = mt[6:7, :]
        ah = mt[7:8, :]

        b1x = jax.nn.sigmoid(hot[0:1, :])
        b1y = jax.nn.sigmoid(hot[1:2, :])
        w1_ = jnp.exp(hot[2:3, :]) * aw
        h1_ = jnp.exp(hot[3:4, :]) * ah
        ps4 = hot[4:5, :]
        pstc = hot[5:6, :]

        b1x1 = b1x - w1_ / 2
        b1x2 = b1x + w1_ / 2
        b1y1 = b1y - h1_ / 2
        b1y2 = b1y + h1_ / 2
        b2x1 = tbx - tbw / 2
        b2x2 = tbx + tbw / 2
        b2y1 = tby - tbh / 2
        b2y2 = tby + tbh / 2
        inter = (jnp.maximum(jnp.minimum(b1x2, b2x2) -
                             jnp.maximum(b1x1, b2x1), 0.0) *
                 jnp.maximum(jnp.minimum(b1y2, b2y2) -
                             jnp.maximum(b1y1, b2y1), 0.0))
        union = w1_ * h1_ + tbw * tbh - inter + 1e-16
        iou0 = inter / union
        cw = jnp.maximum(b1x2, b2x2) - jnp.minimum(b1x1, b2x1)
        ch = jnp.maximum(b1y2, b2y2) - jnp.minimum(b1y1, b2y1)
        c2_ = cw * cw + ch * ch + 1e-16
        rho2 = ((b2x1 + b2x2 - b1x1 - b1x2) ** 2 +
                (b2y1 + b2y2 - b1y1 - b1y2) ** 2) / 4
        v = (4.0 / 3.14159 ** 2) * (_atan_pos(tbw / tbh) -
                                    _atan_pos(w1_ / h1_)) ** 2
        alpha = v / (v - iou0 + (1.0 + 1e-16))
        iou = iou0 - (rho2 / c2_ + v * alpha)

        box_p = jnp.sum((1.0 - iou) * valid)
        cnt_p = jnp.sum(valid)

        cw = cb[...]
        lowf = lax.bitcast_convert_type(cw << 16, jnp.float32)
        highf = lax.bitcast_convert_type(cw & jnp.int32(-65536), jnp.float32)
        cbf = jnp.concatenate([lowf, highf], axis=1)
        sp = jnp.logaddexp(0.0, cbf)
        s0 = jnp.sum(sp[:, 5:85], axis=1, keepdims=True)
        s1 = jnp.sum(sp[:, 90:170], axis=1, keepdims=True)
        s2 = jnp.sum(sp[:, 175:255], axis=1, keepdims=True)
        scol = jnp.where(a_dyn == 0, s0, jnp.where(a_dyn == 1, s1, s2))
        cls_p = jnp.dot(valid, scol)[0, 0] - jnp.sum(pstc * valid)

        wsum = jnp.sum(w[...], axis=0, keepdims=True)
        win_p = jnp.sum(wsum * jnp.maximum(iou, 0.0) * ps4)

        vals = jnp.where(c_i == 0, box_p,
                         jnp.where(c_i == 1, cnt_p,
                                   jnp.where(c_i == 2, cls_p, win_p)))
        acc += jnp.where((r_i == s_) & (c_i < 4), vals, 0.0)

    out_ref[...] += acc


def _math(clss, hots, metas, win):
    nblk = 15
    bw = _N // nblk
    return pl.pallas_call(
        _math_body,
        grid=(nblk,),
        out_shape=jax.ShapeDtypeStruct((8, 128), jnp.float32),
        in_specs=(
            [pl.BlockSpec((bw, 128), lambda i: (i, 0)) for _ in range(3)] +
            [pl.BlockSpec((8, bw), lambda i: (0, i)) for _ in range(3)] +
            [pl.BlockSpec((8, bw), lambda i: (0, i)) for _ in range(3)] +
            [pl.BlockSpec((_NW, bw), lambda i, s_=s_: (0, s_ * nblk + i))
             for s_ in range(3)]),
        out_specs=pl.BlockSpec((8, 128), lambda i: (0, 0)),
    )(*clss, *hots, *metas, win, win, win)


def kernel(pred0, pred1, pred2, targets):
    preds = (pred0, pred1, pred2)
    tt = jnp.pad(targets.T, ((0, 2), (0, 0)))

    auxs, metas = [], []
    for s in range(3):
        aux, meta = _prep(tt, s)
        auxs.append(aux)
        metas.append(meta)

    clss, hots, objs = [None] * 3, [None] * 3, [None] * 3
    views = [jnp.transpose(preds[s], _PERMS[s]) for s in range(3)]
    tab2, objs[2] = _repack(views[2], 2)
    clss[2], hots[2], win = _sc_g2d(tab2, auxs[2], metas)
    tab1, objs[1] = _repack(views[1], 1, chain=objs[2])
    clss[1], hots[1] = _sc_gather(tab1, auxs[1])
    tab0, objs[0] = _repack(views[0], 0, chain=objs[1])
    clss[0], hots[0] = _sc_gather(tab0, auxs[0])

    res = _math(tuple(clss), tuple(hots), metas, win)

    lbox = jnp.float32(0.0)
    lobj = jnp.float32(0.0)
    lcls = jnp.float32(0.0)
    for s in range(3):
        H, W = _HWS[s]
        box_p, cnt, cls_p, win_p = res[s, 0], res[s, 1], res[s, 2], res[s, 3]
        lbox += box_p / cnt
        lcls += cls_p / (cnt * _NC)
        lobj += (objs[s][0, 0] - win_p) / (_NB * 3 * H * W)
    lbox *= 0.05
    lcls *= 0.5
    loss = lbox + lobj + lcls
    return loss, jnp.stack([lbox, lobj, lcls])


# speedup vs baseline: 1.0804x; 1.0804x over previous
import functools

import numpy as np
import jax
import jax.numpy as jnp
from jax import lax
from jax.experimental import pallas as pl
from jax.experimental.pallas import tpu as pltpu
from jax.experimental.pallas import tpu_sc as plsc

_NC = 80
_IMG = 640
_NB = 32
_NT = 1024
_N = 15360
_ANCH = np.array(
    [[10.0, 13.0], [16.0, 30.0], [33.0, 23.0], [30.0, 61.0], [62.0, 45.0],
     [59.0, 119.0], [116.0, 90.0], [156.0, 198.0], [373.0, 326.0]],
    dtype=np.float32)
_HWS = [(80, 80), (40, 40), (20, 20)]
_CELL_BASE = [0, _NB * 3 * 6400, _NB * 3 * 6400 + _NB * 3 * 1600]
_DTOT = _NB * 3 * (6400 + 1600 + 400)
_SENT = 4.0e6

_NW = 32
_DCH = _DTOT // _NW
_CK = 128
_PERMS = [(0, 2, 3, 1), (0, 2, 3, 1), (2, 3, 0, 1)]


def _repack_body(*refs, bh, W):
    p_ref, tab_ref, obj_ref = refs[0], refs[-2], refs[-1]
    i = pl.program_id(0)

    @pl.when(i == 0)
    def _():
        obj_ref[...] = jnp.zeros_like(obj_ref)

    x = p_ref[0]
    x2 = x.reshape(bh * W, 255)
    xp = jnp.concatenate(
        [x2, jnp.zeros((bh * W, 1), jnp.float32)], axis=1)
    b = lax.bitcast_convert_type(xp, jnp.int32)
    tab_ref[...] = ((b[:, :128] >> 16) & 0xFFFF) | (b[:, 128:256] &
                                                    jnp.int32(-65536))
    s = (jnp.sum(jnp.logaddexp(0.0, x2[:, 4:5])) +
         jnp.sum(jnp.logaddexp(0.0, x2[:, 89:90])) +
         jnp.sum(jnp.logaddexp(0.0, x2[:, 174:175])))
    r = lax.broadcasted_iota(jnp.int32, (8, 128), 0)
    c = lax.broadcasted_iota(jnp.int32, (8, 128), 1)
    obj_ref[...] += jnp.where((r == 0) & (c == 0), s, 0.0)


def _repack(p_cl, s, chain=None):
    d0, d1, d2 = p_cl.shape[0], p_cl.shape[1], p_cl.shape[2]
    R = d0 * d1 * d2
    extra = [] if chain is None else [chain]
    return pl.pallas_call(
        functools.partial(_repack_body, bh=d1, W=d2),
        grid=(d0,),
        out_shape=[jax.ShapeDtypeStruct((R, 128), jnp.int32),
                   jax.ShapeDtypeStruct((8, 128), jnp.float32)],
        in_specs=([pl.BlockSpec((1, d1, d2, 255), lambda i: (i, 0, 0, 0))] +
                  [pl.BlockSpec((8, 128), lambda i: (0, 0))
                   for _ in extra]),
        out_specs=[pl.BlockSpec((d1 * d2, 128), lambda i: (i, 0)),
                   pl.BlockSpec((8, 128), lambda i: (0, 0))],
    )(p_cl, *extra)


def _prep_body(tt_ref, aux_ref, meta_ref, *, H, W, anchors, cell_base, border):
    col = lax.broadcasted_iota(jnp.int32, (1, _N), 1)
    a = (col // _NT) % 3
    o = col // (3 * _NT)

    def tiled(r):
        row = tt_ref[r:r + 1, :]
        return jnp.concatenate([row] * 15, axis=1)

    bi = tiled(0)
    cls_f = tiled(1)
    gx = tiled(2) * W
    gy = tiled(3) * H
    gw = tiled(4) * W
    gh = tiled(5) * H

    af = a.astype(jnp.float32)
    aw = jnp.where(af == 0.0, anchors[0, 0],
                   jnp.where(af == 1.0, anchors[1, 0], anchors[2, 0]))
    ah = jnp.where(af == 0.0, anchors[0, 1],
                   jnp.where(af == 1.0, anchors[1, 1], anchors[2, 1]))
    rw = gw / aw
    rh = gh / ah
    fitf = jnp.where(
        jnp.maximum(jnp.maximum(rw, 1.0 / rw), jnp.maximum(rh, 1.0 / rh)) < 4.0,
        1.0, 0.0)
    gxi = W - gx
    gyi = H - gy

    def near(u):
        return jnp.where(u % 1.0 < 0.5, 1.0, 0.0) * jnp.where(u > 1.0, 1.0, 0.0)

    jk0, jk1, lm0, lm1 = near(gx), near(gy), near(gxi), near(gyi)
    jmf = jnp.where(o == 0, 1.0,
                    jnp.where(o == 1, jk0,
                              jnp.where(o == 2, jk1,
                                        jnp.where(o == 3, lm0, lm1))))
    validf = jmf * fitf
    valid = validf > 0.5
    ox = jnp.where(o == 1, 1.0, jnp.where(o == 3, -1.0, 0.0))
    oy = jnp.where(o == 2, 1.0, jnp.where(o == 4, -1.0, 0.0))
    gi0 = (gx - ox).astype(jnp.int32)
    gj0 = (gy - oy).astype(jnp.int32)
    gi = jnp.clip(gi0, 0, W - 1)
    gj = jnp.clip(gj0, 0, H - 1)
    bii = bi.astype(jnp.int32)
    clsi = cls_f.astype(jnp.int32)

    if border:
        rowidx = (gj * W + gi) * _NB + bii
    else:
        rowidx = (bii * H + gj) * W + gi
    zero = jnp.zeros((1, _N), jnp.int32)
    aux_ref[...] = jnp.concatenate(
        [rowidx, clsi, zero, zero, zero, zero, zero, zero], axis=0)

    cellf = jnp.where(
        valid,
        (((bii * 3 + a) * H + gj) * W + gi + cell_base).astype(jnp.float32),
        _SENT)
    meta_ref[...] = jnp.concatenate(
        [gx - gi0.astype(jnp.float32), gy - gj0.astype(jnp.float32),
         gw, gh, validf, cellf,
         jnp.broadcast_to(aw, (1, _N)), jnp.broadcast_to(ah, (1, _N))],
        axis=0)


def _prep(tt_pad, s):
    H, W = _HWS[s]
    stride = _IMG // W
    anchors = _ANCH[s * 3:(s + 1) * 3] / stride
    return pl.pallas_call(
        functools.partial(_prep_body, H=H, W=W, anchors=anchors,
                          cell_base=_CELL_BASE[s], border=(s == 2)),
        out_shape=[
            jax.ShapeDtypeStruct((8, _N), jnp.int32),
            jax.ShapeDtypeStruct((8, _N), jnp.float32),
        ],
        in_specs=[pl.BlockSpec((8, _NT), lambda: (0, 0))],
        out_specs=[pl.BlockSpec((8, _N), lambda: (0, 0)),
                   pl.BlockSpec((8, _N), lambda: (0, 0))],
    )(tt_pad)


def _sc_gather_body(tab, aux, clsout, hot,
                    rowbuf, clsbuf, databuf, hotbuf, gsem):
    wid = lax.axis_index("s") * 2 + lax.axis_index("c")
    lane = lax.iota(jnp.int32, 16)
    trips = (120 - wid + 31) // 32

    def chunk(t_, _):
        ci = wid + 32 * t_
        a_ = (ci // 8) % 3
        col0 = pl.multiple_of(ci * _CK, 128)
        pltpu.sync_copy(aux.at[0, pl.ds(col0, _CK)], rowbuf)
        pltpu.sync_copy(aux.at[1, pl.ds(col0, _CK)], clsbuf)
        pltpu.async_copy(tab.at[rowbuf], databuf, gsem).wait()

        def getchan(q, c):
            w = plsc.load_gather(databuf, [q, c % 128])
            bits = jnp.where(c >= 128,
                             w & jnp.int32(-65536), w << 16)
            return plsc.bitcast(bits, jnp.float32)

        def sub(i, _):
            q = i * 16 + lane
            for ch in range(5):
                c = a_ * 85 + jnp.full((16,), ch, jnp.int32)
                hotbuf[ch, pl.ds(i * 16, 16)] = getchan(q, c)
            cv = clsbuf[pl.ds(i * 16, 16)]
            hotbuf[5, pl.ds(i * 16, 16)] = getchan(q, a_ * 85 + 5 + cv)
            return _

        lax.fori_loop(0, _CK // 16, sub, 0)
        pltpu.sync_copy(databuf, clsout.at[pl.ds(col0, _CK), :])
        pltpu.sync_copy(hotbuf, hot.at[:, pl.ds(col0, _CK)])
        return _

    lax.fori_loop(0, trips, chunk, 0)


def _sc_dedup_body(m0, m1, m2, win, cellbuf, dense, winbuf):
    wid = lax.axis_index("s") * 2 + lax.axis_index("c")
    metas = (m0, m1, m2)
    lane = lax.iota(jnp.int32, 16)

    def ms(i, _):
        dense[pl.ds(i * 16, 16)] = jnp.full((16,), -1, jnp.int32)
        return _

    lax.fori_loop(0, _DCH // 16, ms, 0, unroll=4)

    wbase = wid * _DCH
    for s_ in range(3):
        pltpu.sync_copy(metas[s_].at[5, :], cellbuf)

        def p1b(i, _):
            c = cellbuf[pl.ds(i * 16, 16)].astype(jnp.int32) - wbase
            m = (c >= 0) & (c < _DCH)
            cs = jnp.where(m, c, 0)
            plsc.store_scatter(dense, [cs], i * 16 + lane, mask=m)
            return _

        lax.fori_loop(0, _N // 16, p1b, 0, unroll=4)

        def p2b(i, _):
            c = cellbuf[pl.ds(i * 16, 16)].astype(jnp.int32) - wbase
            m = (c >= 0) & (c < _DCH)
            cs = jnp.where(m, c, 0)
            w = plsc.load_gather(dense, [cs], mask=m)
            isw = m & (w == i * 16 + lane)
            winbuf[pl.ds(i * 16, 16)] = jnp.where(isw, 1.0, 0.0)
            return _

        lax.fori_loop(0, _N // 16, p2b, 0, unroll=4)
        pltpu.sync_copy(winbuf, win.at[wid, pl.ds(s_ * _N, _N)])


def _sc_mesh():
    return plsc.VectorSubcoreMesh(core_axis_name="c", subcore_axis_name="s",
                                  num_cores=2, num_subcores=16)


def _sc_g2d_body(tab, aux, m0, m1, m2, clsout, hot, win,
                 rowbuf, clsbuf, databuf, hotbuf, cellbuf, dense, winbuf,
                 gsem):
    _sc_gather_body(tab, aux, clsout, hot, rowbuf, clsbuf, databuf, hotbuf,
                    gsem)
    _sc_dedup_body(m0, m1, m2, win, cellbuf, dense, winbuf)


def _sc_g2d(tab, aux, metas):
    f = pl.kernel(
        _sc_g2d_body,
        out_type=[
            jax.ShapeDtypeStruct((_N, 128), jnp.int32),
            jax.ShapeDtypeStruct((8, _N), jnp.float32),
            jax.ShapeDtypeStruct((_NW, 3 * _N), jnp.float32),
        ],
        mesh=_sc_mesh(),
        scratch_types=[
            pltpu.VMEM((_CK,), jnp.int32),
            pltpu.VMEM((_CK,), jnp.int32),
            pltpu.VMEM((_CK, 128), jnp.int32),
            pltpu.VMEM((8, _CK), jnp.float32),
            pltpu.VMEM((_N,), jnp.float32),
            pltpu.VMEM((_DCH,), jnp.int32),
            pltpu.VMEM((_N,), jnp.float32),
            pltpu.SemaphoreType.DMA,
        ],
        compiler_params=pltpu.CompilerParams(needs_layout_passes=False),
    )
    return f(tab, aux, *metas)


def _sc_gather(tab, aux):
    f = pl.kernel(
        _sc_gather_body,
        out_type=[
            jax.ShapeDtypeStruct((_N, 128), jnp.int32),
            jax.ShapeDtypeStruct((8, _N), jnp.float32),
        ],
        mesh=_sc_mesh(),
        scratch_types=[
            pltpu.VMEM((_CK,), jnp.int32),
            pltpu.VMEM((_CK,), jnp.int32),
            pltpu.VMEM((_CK, 128), jnp.int32),
            pltpu.VMEM((8, _CK), jnp.float32),
            pltpu.SemaphoreType.DMA,
        ],
        compiler_params=pltpu.CompilerParams(needs_layout_passes=False),
    )
    return f(tab, aux)


def _atan_pos(x):
    inv = x > 1.0
    y = jnp.where(inv, 1.0 / x, x)
    z = y * y
    p = y * (0.9998660 + z * (-0.3302995 + z * (0.1801410 + z *
             (-0.0851330 + z * 0.0208351))))
    return jnp.where(inv, (np.pi / 2) - p, p)


def _math_body(c0, c1, c2, h0, h1, h2, m0, m1, m2, w0, w1, w2, out_ref):
    i = pl.program_id(0)

    @pl.when(i == 0)
    def _():
        out_ref[...] = jnp.zeros_like(out_ref)

    a_dyn = i % 3
    acc = jnp.zeros((8, 128), jnp.float32)
    r_i = lax.broadcasted_iota(jnp.int32, (8, 128), 0)
    c_i = lax.broadcasted_iota(jnp.int32, (8, 128), 1)
    for s_, (cb, hot, mt, w) in enumerate(((c0, h0, m0, w0), (c1, h1, m1, w1),
                                           (c2, h2, m2, w2))):
        tbx = mt[0:1, :]
        tby = mt[1:2, :]
        tbw = mt[2:3, :]
        tbh = mt[3:4, :]
        valid = mt[4:5, :]
        aw = mt[6:7, :]
        ah = mt[7:8, :]

        b1x = jax.nn.sigmoid(hot[0:1, :])
        b1y = jax.nn.sigmoid(hot[1:2, :])
        w1_ = jnp.exp(hot[2:3, :]) * aw
        h1_ = jnp.exp(hot[3:4, :]) * ah
        ps4 = hot[4:5, :]
        pstc = hot[5:6, :]

        b1x1 = b1x - w1_ / 2
        b1x2 = b1x + w1_ / 2
        b1y1 = b1y - h1_ / 2
        b1y2 = b1y + h1_ / 2
        b2x1 = tbx - tbw / 2
        b2x2 = tbx + tbw / 2
        b2y1 = tby - tbh / 2
        b2y2 = tby + tbh / 2
        inter = (jnp.maximum(jnp.minimum(b1x2, b2x2) -
                             jnp.maximum(b1x1, b2x1), 0.0) *
                 jnp.maximum(jnp.minimum(b1y2, b2y2) -
                             jnp.maximum(b1y1, b2y1), 0.0))
        union = w1_ * h1_ + tbw * tbh - inter + 1e-16
        iou0 = inter / union
        cw = jnp.maximum(b1x2, b2x2) - jnp.minimum(b1x1, b2x1)
        ch = jnp.maximum(b1y2, b2y2) - jnp.minimum(b1y1, b2y1)
        c2_ = cw * cw + ch * ch + 1e-16
        rho2 = ((b2x1 + b2x2 - b1x1 - b1x2) ** 2 +
                (b2y1 + b2y2 - b1y1 - b1y2) ** 2) / 4
        v = (4.0 / 3.14159 ** 2) * (_atan_pos(tbw / tbh) -
                                    _atan_pos(w1_ / h1_)) ** 2
        alpha = v / (v - iou0 + (1.0 + 1e-16))
        iou = iou0 - (rho2 / c2_ + v * alpha)

        box_p = jnp.sum((1.0 - iou) * valid)
        cnt_p = jnp.sum(valid)

        cw = cb[...]
        lowf = lax.bitcast_convert_type(cw << 16, jnp.float32)
        highf = lax.bitcast_convert_type(cw & jnp.int32(-65536), jnp.float32)
        cbf = jnp.concatenate([lowf, highf], axis=1)
        sp = jnp.logaddexp(0.0, cbf)
        s0 = jnp.sum(sp[:, 5:85], axis=1, keepdims=True)
        s1 = jnp.sum(sp[:, 90:170], axis=1, keepdims=True)
        s2 = jnp.sum(sp[:, 175:255], axis=1, keepdims=True)
        scol = jnp.where(a_dyn == 0, s0, jnp.where(a_dyn == 1, s1, s2))
        cls_p = jnp.dot(valid, scol)[0, 0] - jnp.sum(pstc * valid)

        wsum = jnp.sum(w[...], axis=0, keepdims=True)
        win_p = jnp.sum(wsum * jnp.maximum(iou, 0.0) * ps4)

        vals = jnp.where(c_i == 0, box_p,
                         jnp.where(c_i == 1, cnt_p,
                                   jnp.where(c_i == 2, cls_p, win_p)))
        acc += jnp.where((r_i == s_) & (c_i < 4), vals, 0.0)

    out_ref[...] += acc


def _math(clss, hots, metas, win):
    nblk = 15
    bw = _N // nblk
    return pl.pallas_call(
        _math_body,
        grid=(nblk,),
        out_shape=jax.ShapeDtypeStruct((8, 128), jnp.float32),
        in_specs=(
            [pl.BlockSpec((bw, 128), lambda i: (i, 0)) for _ in range(3)] +
            [pl.BlockSpec((8, bw), lambda i: (0, i)) for _ in range(3)] +
            [pl.BlockSpec((8, bw), lambda i: (0, i)) for _ in range(3)] +
            [pl.BlockSpec((_NW, bw), lambda i, s_=s_: (0, s_ * nblk + i))
             for s_ in range(3)]),
        out_specs=pl.BlockSpec((8, 128), lambda i: (0, 0)),
    )(*clss, *hots, *metas, win, win, win)


def kernel(pred0, pred1, pred2, targets):
    preds = (pred0, pred1, pred2)
    tt = jnp.pad(targets.T, ((0, 2), (0, 0)))

    auxs, metas = [], []
    for s in range(3):
        aux, meta = _prep(tt, s)
        auxs.append(aux)
        metas.append(meta)

    clss, hots, objs = [None] * 3, [None] * 3, [None] * 3
    views = [jnp.transpose(preds[s], _PERMS[s]) for s in range(3)]
    tab2, objs[2] = _repack(views[2], 2)
    clss[2], hots[2], win = _sc_g2d(tab2, auxs[2], metas)
    tab1, objs[1] = _repack(views[1], 1, chain=objs[2])
    clss[1], hots[1] = _sc_gather(tab1, auxs[1])
    tab0, objs[0] = _repack(views[0], 0, chain=objs[1])
    clss[0], hots[0] = _sc_gather(tab0, auxs[0])

    res = _math(tuple(clss), tuple(hots), metas, win)

    lbox = jnp.float32(0.0)
    lobj = jnp.float32(0.0)
    lcls = jnp.float32(0.0)
    for s in range(3):
        H, W = _HWS[s]
        box_p, cnt, cls_p, win_p = res[s, 0], res[s, 1], res[s, 2], res[s, 3]
        lbox += box_p / cnt
        lcls += cls_p / (cnt * _NC)
        lobj += (objs[s][0, 0] - win_p) / (_NB * 3 * H * W)
    lbox *= 0.05
    lcls *= 0.5
    loss = lbox + lobj + lcls
    return loss, jnp.stack([lbox, lobj, lcls])


# speedup vs baseline: 1.0829x; 1.0023x over previous
import functools

import numpy as np
import jax
import jax.numpy as jnp
from jax import lax
from jax.experimental import pallas as pl
from jax.experimental.pallas import tpu as pltpu
from jax.experimental.pallas import tpu_sc as plsc

_NC = 80
_IMG = 640
_NB = 32
_NT = 1024
_N = 15360
_ANCH = np.array(
    [[10.0, 13.0], [16.0, 30.0], [33.0, 23.0], [30.0, 61.0], [62.0, 45.0],
     [59.0, 119.0], [116.0, 90.0], [156.0, 198.0], [373.0, 326.0]],
    dtype=np.float32)
_HWS = [(80, 80), (40, 40), (20, 20)]
_CELL_BASE = [0, _NB * 3 * 6400, _NB * 3 * 6400 + _NB * 3 * 1600]
_DTOT = _NB * 3 * (6400 + 1600 + 400)
_SENT = 4.0e6

_NW = 32
_DCH = _DTOT // _NW
_CK = 128
_PERMS = [(0, 2, 3, 1), (0, 2, 3, 1), (2, 3, 0, 1)]


def _repack_body(*refs, bh, W):
    p_ref, tab_ref, obj_ref = refs[0], refs[-2], refs[-1]
    i = pl.program_id(0)

    @pl.when(i == 0)
    def _():
        obj_ref[...] = jnp.zeros_like(obj_ref)

    x = p_ref[0]
    x2 = x.reshape(bh * W, 255)
    xp = jnp.concatenate(
        [x2, jnp.zeros((bh * W, 1), jnp.float32)], axis=1)
    b = lax.bitcast_convert_type(xp, jnp.int32)
    tab_ref[...] = ((b[:, :128] >> 16) & 0xFFFF) | (b[:, 128:256] &
                                                    jnp.int32(-65536))
    s = (jnp.sum(jnp.logaddexp(0.0, x2[:, 4:5])) +
         jnp.sum(jnp.logaddexp(0.0, x2[:, 89:90])) +
         jnp.sum(jnp.logaddexp(0.0, x2[:, 174:175])))
    r = lax.broadcasted_iota(jnp.int32, (8, 128), 0)
    c = lax.broadcasted_iota(jnp.int32, (8, 128), 1)
    obj_ref[...] += jnp.where((r == 0) & (c == 0), s, 0.0)


def _repack(p_cl, s, chain=None):
    d0, d1, d2 = p_cl.shape[0], p_cl.shape[1], p_cl.shape[2]
    R = d0 * d1 * d2
    extra = [] if chain is None else [chain]
    return pl.pallas_call(
        functools.partial(_repack_body, bh=d1, W=d2),
        grid=(d0,),
        out_shape=[jax.ShapeDtypeStruct((R, 128), jnp.int32),
                   jax.ShapeDtypeStruct((8, 128), jnp.float32)],
        in_specs=([pl.BlockSpec((1, d1, d2, 255), lambda i: (i, 0, 0, 0))] +
                  [pl.BlockSpec((8, 128), lambda i: (0, 0))
                   for _ in extra]),
        out_specs=[pl.BlockSpec((d1 * d2, 128), lambda i: (i, 0)),
                   pl.BlockSpec((8, 128), lambda i: (0, 0))],
    )(p_cl, *extra)


def _prep_body(tt_ref, aux_ref, meta_ref, *, H, W, anchors, cell_base, border):
    col = lax.broadcasted_iota(jnp.int32, (1, _N), 1)
    a = (col // _NT) % 3
    o = col // (3 * _NT)

    def tiled(r):
        row = tt_ref[r:r + 1, :]
        return jnp.concatenate([row] * 15, axis=1)

    bi = tiled(0)
    cls_f = tiled(1)
    gx = tiled(2) * W
    gy = tiled(3) * H
    gw = tiled(4) * W
    gh = tiled(5) * H

    af = a.astype(jnp.float32)
    aw = jnp.where(af == 0.0, anchors[0, 0],
                   jnp.where(af == 1.0, anchors[1, 0], anchors[2, 0]))
    ah = jnp.where(af == 0.0, anchors[0, 1],
                   jnp.where(af == 1.0, anchors[1, 1], anchors[2, 1]))
    rw = gw / aw
    rh = gh / ah
    fitf = jnp.where(
        jnp.maximum(jnp.maximum(rw, 1.0 / rw), jnp.maximum(rh, 1.0 / rh)) < 4.0,
        1.0, 0.0)
    gxi = W - gx
    gyi = H - gy

    def near(u):
        return jnp.where(u % 1.0 < 0.5, 1.0, 0.0) * jnp.where(u > 1.0, 1.0, 0.0)

    jk0, jk1, lm0, lm1 = near(gx), near(gy), near(gxi), near(gyi)
    jmf = jnp.where(o == 0, 1.0,
                    jnp.where(o == 1, jk0,
                              jnp.where(o == 2, jk1,
                                        jnp.where(o == 3, lm0, lm1))))
    validf = jmf * fitf
    valid = validf > 0.5
    ox = jnp.where(o == 1, 1.0, jnp.where(o == 3, -1.0, 0.0))
    oy = jnp.where(o == 2, 1.0, jnp.where(o == 4, -1.0, 0.0))
    gi0 = (gx - ox).astype(jnp.int32)
    gj0 = (gy - oy).astype(jnp.int32)
    gi = jnp.clip(gi0, 0, W - 1)
    gj = jnp.clip(gj0, 0, H - 1)
    bii = bi.astype(jnp.int32)
    clsi = cls_f.astype(jnp.int32)

    if border:
        rowidx = (gj * W + gi) * _NB + bii
    else:
        rowidx = (bii * H + gj) * W + gi
    zero = jnp.zeros((1, _N), jnp.int32)
    aux_ref[...] = jnp.concatenate(
        [rowidx, clsi, zero, zero, zero, zero, zero, zero], axis=0)

    cellf = jnp.where(
        valid,
        (((bii * 3 + a) * H + gj) * W + gi + cell_base).astype(jnp.float32),
        _SENT)
    meta_ref[...] = jnp.concatenate(
        [gx - gi0.astype(jnp.float32), gy - gj0.astype(jnp.float32),
         gw, gh, validf, cellf,
         jnp.broadcast_to(aw, (1, _N)), jnp.broadcast_to(ah, (1, _N))],
        axis=0)


def _prep(tt_pad, s):
    H, W = _HWS[s]
    stride = _IMG // W
    anchors = _ANCH[s * 3:(s + 1) * 3] / stride
    return pl.pallas_call(
        functools.partial(_prep_body, H=H, W=W, anchors=anchors,
                          cell_base=_CELL_BASE[s], border=(s == 2)),
        out_shape=[
            jax.ShapeDtypeStruct((8, _N), jnp.int32),
            jax.ShapeDtypeStruct((8, _N), jnp.float32),
        ],
        in_specs=[pl.BlockSpec((8, _NT), lambda: (0, 0))],
        out_specs=[pl.BlockSpec((8, _N), lambda: (0, 0)),
                   pl.BlockSpec((8, _N), lambda: (0, 0))],
    )(tt_pad)


def _sc_gather_body(tab, aux, clsout, hot,
                    rowbuf, clsbuf, databuf, hotbuf, gsem):
    wid = lax.axis_index("s") * 2 + lax.axis_index("c")
    lane = lax.iota(jnp.int32, 16)
    trips = (120 - wid + 31) // 32

    def chunk(t_, _):
        ci = wid + 32 * t_
        a_ = (ci // 8) % 3
        col0 = pl.multiple_of(ci * _CK, 128)
        pltpu.sync_copy(aux.at[0, pl.ds(col0, _CK)], rowbuf)
        pltpu.sync_copy(aux.at[1, pl.ds(col0, _CK)], clsbuf)
        pltpu.async_copy(tab.at[rowbuf], databuf, gsem).wait()

        def getchan(q, c):
            w = plsc.load_gather(databuf, [q, c % 128])
            bits = jnp.where(c >= 128,
                             w & jnp.int32(-65536), w << 16)
            return plsc.bitcast(bits, jnp.float32)

        def sub(i, _):
            q = i * 16 + lane
            for ch in range(5):
                c = a_ * 85 + jnp.full((16,), ch, jnp.int32)
                hotbuf[ch, pl.ds(i * 16, 16)] = getchan(q, c)
            cv = clsbuf[pl.ds(i * 16, 16)]
            hotbuf[5, pl.ds(i * 16, 16)] = getchan(q, a_ * 85 + 5 + cv)
            return _

        lax.fori_loop(0, _CK // 16, sub, 0)
        pltpu.sync_copy(databuf, clsout.at[pl.ds(col0, _CK), :])
        pltpu.sync_copy(hotbuf, hot.at[:, pl.ds(col0, _CK)])
        return _

    lax.fori_loop(0, trips, chunk, 0)


def _sc_dedup_body(m0, m1, m2, win, cellbuf, dense, winbuf):
    wid = lax.axis_index("s") * 2 + lax.axis_index("c")
    metas = (m0, m1, m2)
    lane = lax.iota(jnp.int32, 16)

    def ms(i, _):
        dense[pl.ds(i * 16, 16)] = jnp.full((16,), -1, jnp.int32)
        return _

    lax.fori_loop(0, _DCH // 16, ms, 0, unroll=4)

    wbase = wid * _DCH
    for s_ in range(3):
        pltpu.sync_copy(metas[s_].at[5, :], cellbuf)

        def p1b(i, _):
            c = cellbuf[pl.ds(i * 16, 16)].astype(jnp.int32) - wbase
            m = (c >= 0) & (c < _DCH)
            cs = jnp.where(m, c, 0)
            plsc.store_scatter(dense, [cs], i * 16 + lane, mask=m)
            return _

        lax.fori_loop(0, _N // 16, p1b, 0, unroll=4)

        def p2b(i, _):
            c = cellbuf[pl.ds(i * 16, 16)].astype(jnp.int32) - wbase
            m = (c >= 0) & (c < _DCH)
            cs = jnp.where(m, c, 0)
            w = plsc.load_gather(dense, [cs], mask=m)
            isw = m & (w == i * 16 + lane)
            winbuf[pl.ds(i * 16, 16)] = jnp.where(isw, 1.0, 0.0)
            return _

        lax.fori_loop(0, _N // 16, p2b, 0, unroll=4)
        pltpu.sync_copy(winbuf, win.at[wid, pl.ds(s_ * _N, _N)])


def _sc_mesh():
    return plsc.VectorSubcoreMesh(core_axis_name="c", subcore_axis_name="s",
                                  num_cores=2, num_subcores=16)


def _sc_g2d_body(tab, aux, m0, m1, m2, clsout, hot, win,
                 rowbuf, clsbuf, databuf, hotbuf, cellbuf, dense, winbuf,
                 gsem):
    _sc_gather_body(tab, aux, clsout, hot, rowbuf, clsbuf, databuf, hotbuf,
                    gsem)
    _sc_dedup_body(m0, m1, m2, win, cellbuf, dense, winbuf)


def _sc_g2d(tab, aux, metas):
    f = pl.kernel(
        _sc_g2d_body,
        out_type=[
            jax.ShapeDtypeStruct((_N, 128), jnp.int32),
            jax.ShapeDtypeStruct((8, _N), jnp.float32),
            jax.ShapeDtypeStruct((_NW, 3 * _N), jnp.float32),
        ],
        mesh=_sc_mesh(),
        scratch_types=[
            pltpu.VMEM((_CK,), jnp.int32),
            pltpu.VMEM((_CK,), jnp.int32),
            pltpu.VMEM((_CK, 128), jnp.int32),
            pltpu.VMEM((8, _CK), jnp.float32),
            pltpu.VMEM((_N,), jnp.float32),
            pltpu.VMEM((_DCH,), jnp.int32),
            pltpu.VMEM((_N,), jnp.float32),
            pltpu.SemaphoreType.DMA,
        ],
        compiler_params=pltpu.CompilerParams(needs_layout_passes=False),
    )
    return f(tab, aux, *metas)


def _sc_gather(tab, aux):
    f = pl.kernel(
        _sc_gather_body,
        out_type=[
            jax.ShapeDtypeStruct((_N, 128), jnp.int32),
            jax.ShapeDtypeStruct((8, _N), jnp.float32),
        ],
        mesh=_sc_mesh(),
        scratch_types=[
            pltpu.VMEM((_CK,), jnp.int32),
            pltpu.VMEM((_CK,), jnp.int32),
            pltpu.VMEM((_CK, 128), jnp.int32),
            pltpu.VMEM((8, _CK), jnp.float32),
            pltpu.SemaphoreType.DMA,
        ],
        compiler_params=pltpu.CompilerParams(needs_layout_passes=False),
    )
    return f(tab, aux)


def _atan_pos(x):
    inv = x > 1.0
    y = jnp.where(inv, 1.0 / x, x)
    z = y * y
    p = y * (0.9998660 + z * (-0.3302995 + z * (0.1801410 + z *
             (-0.0851330 + z * 0.0208351))))
    return jnp.where(inv, (np.pi / 2) - p, p)


def _math_body(cb, hot, mt, w, out_ref):
    i = pl.program_id(0)

    @pl.when(i == 0)
    def _():
        out_ref[...] = jnp.zeros_like(out_ref)

    a_dyn = i % 3
    r_i = lax.broadcasted_iota(jnp.int32, (8, 128), 0)
    c_i = lax.broadcasted_iota(jnp.int32, (8, 128), 1)

    tbx = mt[0:1, :]
    tby = mt[1:2, :]
    tbw = mt[2:3, :]
    tbh = mt[3:4, :]
    valid = mt[4:5, :]
    aw = mt[6:7, :]
    ah = mt[7:8, :]

    b1x = jax.nn.sigmoid(hot[0:1, :])
    b1y = jax.nn.sigmoid(hot[1:2, :])
    w1_ = jnp.exp(hot[2:3, :]) * aw
    h1_ = jnp.exp(hot[3:4, :]) * ah
    ps4 = hot[4:5, :]
    pstc = hot[5:6, :]

    b1x1 = b1x - w1_ / 2
    b1x2 = b1x + w1_ / 2
    b1y1 = b1y - h1_ / 2
    b1y2 = b1y + h1_ / 2
    b2x1 = tbx - tbw / 2
    b2x2 = tbx + tbw / 2
    b2y1 = tby - tbh / 2
    b2y2 = tby + tbh / 2
    inter = (jnp.maximum(jnp.minimum(b1x2, b2x2) -
                         jnp.maximum(b1x1, b2x1), 0.0) *
             jnp.maximum(jnp.minimum(b1y2, b2y2) -
                         jnp.maximum(b1y1, b2y1), 0.0))
    union = w1_ * h1_ + tbw * tbh - inter + 1e-16
    iou0 = inter / union
    cw = jnp.maximum(b1x2, b2x2) - jnp.minimum(b1x1, b2x1)
    ch = jnp.maximum(b1y2, b2y2) - jnp.minimum(b1y1, b2y1)
    c2_ = cw * cw + ch * ch + 1e-16
    rho2 = ((b2x1 + b2x2 - b1x1 - b1x2) ** 2 +
            (b2y1 + b2y2 - b1y1 - b1y2) ** 2) / 4
    v = (4.0 / 3.14159 ** 2) * (_atan_pos(tbw / tbh) -
                                _atan_pos(w1_ / h1_)) ** 2
    alpha = v / (v - iou0 + (1.0 + 1e-16))
    iou = iou0 - (rho2 / c2_ + v * alpha)

    box_p = jnp.sum((1.0 - iou) * valid)
    cnt_p = jnp.sum(valid)

    cw_ = cb[...]
    lowf = lax.bitcast_convert_type(cw_ << 16, jnp.float32)
    highf = lax.bitcast_convert_type(cw_ & jnp.int32(-65536), jnp.float32)
    cbf = jnp.concatenate([lowf, highf], axis=1)
    sp = jnp.logaddexp(0.0, cbf)
    s0 = jnp.sum(sp[:, 5:85], axis=1, keepdims=True)
    s1 = jnp.sum(sp[:, 90:170], axis=1, keepdims=True)
    s2 = jnp.sum(sp[:, 175:255], axis=1, keepdims=True)
    scol = jnp.where(a_dyn == 0, s0, jnp.where(a_dyn == 1, s1, s2))
    cls_p = jnp.dot(valid, scol)[0, 0] - jnp.sum(pstc * valid)

    wsum = jnp.sum(w[...], axis=0, keepdims=True)
    win_p = jnp.sum(wsum * jnp.maximum(iou, 0.0) * ps4)

    vals = jnp.where(c_i == 0, box_p,
                     jnp.where(c_i == 1, cnt_p,
                               jnp.where(c_i == 2, cls_p, win_p)))
    out_ref[...] += jnp.where((r_i == 0) & (c_i < 4), vals, 0.0)


def _math(cls_s, hot_s, meta_s, win, s):
    nblk = 15
    bw = _N // nblk
    return pl.pallas_call(
        _math_body,
        grid=(nblk,),
        out_shape=jax.ShapeDtypeStruct((8, 128), jnp.float32),
        in_specs=[
            pl.BlockSpec((bw, 128), lambda i: (i, 0)),
            pl.BlockSpec((8, bw), lambda i: (0, i)),
            pl.BlockSpec((8, bw), lambda i: (0, i)),
            pl.BlockSpec((_NW, bw), lambda i, s=s: (0, s * nblk + i)),
        ],
        out_specs=pl.BlockSpec((8, 128), lambda i: (0, 0)),
    )(cls_s, hot_s, meta_s, win)


def kernel(pred0, pred1, pred2, targets):
    preds = (pred0, pred1, pred2)
    tt = jnp.pad(targets.T, ((0, 2), (0, 0)))

    auxs, metas = [], []
    for s in range(3):
        aux, meta = _prep(tt, s)
        auxs.append(aux)
        metas.append(meta)

    clss, hots, objs = [None] * 3, [None] * 3, [None] * 3
    views = [jnp.transpose(preds[s], _PERMS[s]) for s in range(3)]
    tab2, objs[2] = _repack(views[2], 2)
    clss[2], hots[2], win = _sc_g2d(tab2, auxs[2], metas)
    tab1, objs[1] = _repack(views[1], 1, chain=objs[2])
    clss[1], hots[1] = _sc_gather(tab1, auxs[1])
    tab0, objs[0] = _repack(views[0], 0, chain=objs[1])
    clss[0], hots[0] = _sc_gather(tab0, auxs[0])

    ress = [_math(clss[s], hots[s], metas[s], win, s) for s in (2, 1, 0)]
    res = {2: ress[0], 1: ress[1], 0: ress[2]}

    lbox = jnp.float32(0.0)
    lobj = jnp.float32(0.0)
    lcls = jnp.float32(0.0)
    for s in range(3):
        H, W = _HWS[s]
        r_ = res[s]
        box_p, cnt, cls_p, win_p = r_[0, 0], r_[0, 1], r_[0, 2], r_[0, 3]
        lbox += box_p / cnt
        lcls += cls_p / (cnt * _NC)
        lobj += (objs[s][0, 0] - win_p) / (_NB * 3 * H * W)
    lbox *= 0.05
    lcls *= 0.5
    loss = lbox + lobj + lcls
    return loss, jnp.stack([lbox, lobj, lcls])
